# baseline (speedup 1.0000x reference)
import numpy as np

import jax
import jax.numpy as jnp
from jax import lax
from jax.experimental import pallas as pl

_B, _NC, _L, _E = 16, 10, 10, 128
_HIGH_STEPS = 9
_C = 10.0


_ROTS = ((13, 15, 26, 6), (17, 29, 16, 24))


def _threefry2x32(k0, k1, x0, x1):
    x0 = x0.astype(np.uint32).copy()
    x1 = x1.astype(np.uint32).copy()
    ks = (np.uint32(k0), np.uint32(k1),
          np.uint32(k0) ^ np.uint32(k1) ^ np.uint32(0x1BD11BDA))
    x0 += ks[0]
    x1 += ks[1]
    for d in range(5):
        for r in _ROTS[d % 2]:
            x0 += x1
            x1 = ((x1 << np.uint32(r)) | (x1 >> np.uint32(32 - r))).astype(np.uint32)
            x1 ^= x0
        x0 += ks[(d + 1) % 3]
        x1 += ks[(d + 2) % 3] + np.uint32(d + 1)
    return x0, x1


def _np_split(key):
    a, b = _threefry2x32(key[0], key[1],
                         np.zeros(2, np.uint32), np.arange(2, dtype=np.uint32))
    return (a[0], b[0]), (a[1], b[1])


def _np_gumbel(key, shape):
    size = int(np.prod(shape))
    j = np.arange(size, dtype=np.uint64)
    hi = (j >> np.uint64(32)).astype(np.uint32)
    lo = (j & np.uint64(0xFFFFFFFF)).astype(np.uint32)
    a, b = _threefry2x32(key[0], key[1], hi, lo)
    bits = a ^ b
    f = ((bits >> np.uint32(9)) | np.uint32(0x3F800000)).view(np.float32) - np.float32(1.0)
    tiny = np.float32(np.finfo(np.float32).tiny)
    u = np.maximum(tiny, f * (np.float32(1.0) - tiny) + tiny)
    return (-np.log(-np.log(u))).astype(np.float32).reshape(shape)


_GUMBEL_TABLES = None


def _gumbel_tables():
    global _GUMBEL_TABLES
    if _GUMBEL_TABLES is not None:
        return _GUMBEL_TABLES
    key = (np.uint32(0), np.uint32(42))
    gh = np.zeros((_HIGH_STEPS, _B, _NC), np.float32)
    gl = np.zeros((_HIGH_STEPS, _B, _L, _NC), np.float32)
    for i in range(_HIGH_STEPS):
        key, sk = _np_split(key)
        gh[i] = _np_gumbel(sk, (_B, _NC))
        for bid in range(_B):
            key, sk2 = _np_split(key)
            lk = sk2
            for s in range(_L):
                lk, sks = _np_split(lk)
                gl[i, bid, s] = _np_gumbel(sks, (1, _NC))[0]
    _GUMBEL_TABLES = (gh, gl)
    return _GUMBEL_TABLES


_HB = 8


def _decoder_kernel(
    nc_ref, ox_ref, oy_ref, cc_ref, hm_ref, lm_ref,
    whc_ref, bhc_ref, wva_ref, wvb_ref, bvw_ref, initw_ref,
    hwq_ref, hbq_ref, hwk_ref, hbk_ref, hv_ref,
    wlh_ref, blh_ref, wlva_ref, wlvb_ref, blv_ref, linitw_ref,
    lwq_ref, lbq_ref, lwk_ref, lbk_ref, lv_ref,
    gh_ref, gl_ref,
    out_ref,
):
    f32 = jnp.float32
    iota = lax.broadcasted_iota(jnp.int32, (_HB, _NC), 1)

    def matmul(a, w):
        return jnp.dot(a, w, preferred_element_type=f32)

    def argmax_onehot(z):
        m = jnp.max(z, axis=-1, keepdims=True)
        cand = jnp.where(z >= m, iota, 10000)
        idx = jnp.min(cand, axis=-1, keepdims=True)
        return (iota == idx).astype(f32)

    def probs_from_q(q, k, v, mask):
        e = jnp.tanh(q[:, None, :] + k)
        ev = jnp.reshape(matmul(jnp.reshape(e, (_HB * _NC, _E)), v),
                         (_HB, _NC))
        u = _C * jnp.tanh(ev) - 1e8 * mask
        um = jnp.max(u, axis=-1, keepdims=True)
        ex = jnp.exp(u - um)
        return ex / jnp.sum(ex, axis=-1, keepdims=True)

    def row_gather(oh, table):
        acc = None
        for c in range(table.shape[1]):
            w = oh[:, c:c + 1]
            sl = table[:, c]
            if sl.ndim == 3:
                w = w[:, :, None]
            term = w * sl
            acc = term if acc is None else acc + term
        return acc

    hv, lv = hv_ref[...], lv_ref[...]
    wva, wvb, bvw = wva_ref[...], wvb_ref[...], bvw_ref[...]
    wlva, wlvb, blv = wlva_ref[...], wlvb_ref[...], blv_ref[...]
    hwq, hbq = hwq_ref[...], hbq_ref[...]
    lwq, lbq = lwq_ref[...], lbq_ref[...]
    hwk, hbk = hwk_ref[...], hbk_ref[...]
    lwk, lbk = lwk_ref[...], lbk_ref[...]
    whc, bhc = whc_ref[...], bhc_ref[...]
    wlh, blh = wlh_ref[...], blh_ref[...]

    wvf = jnp.concatenate([wva, wvb], axis=0)
    wlvf = jnp.concatenate([wlva, wlvb], axis=0)
    initw = initw_ref[...]
    h_rest0 = matmul(jnp.concatenate([initw[0:1], initw[1:2]], axis=1),
                     wvf) + bvw
    linitw = linitw_ref[...]
    lh_rest0 = matmul(jnp.concatenate([linitw[0:1], linitw[1:2]], axis=1),
                      wlvf) + blv

    nc_full = nc_ref[...]
    ox_full, oy_full = ox_ref[...], oy_ref[...]
    lm_full = lm_ref[...]
    cc_full, hm_full = cc_ref[...], hm_ref[...]
    gh_full, gl_full = gh_ref[...], gl_ref[...]

    total_lp = f32(0.0)
    total_reward = f32(0.0)

    for hb in range(_B // _HB):
        S = slice(hb * _HB, (hb + 1) * _HB)
        cc, hm = cc_full[S], hm_full[S]
        nc, ox, oy, lm = nc_full[S], ox_full[S], oy_full[S], lm_full[S]

        h_mean = jnp.mean(cc, axis=1)
        h_bar = matmul(h_mean, whc) + bhc
        hmask = jnp.where(iota == 0, 1.0, hm)
        cc2d = jnp.reshape(cc, (_HB * _NC, _E))
        k_high = jnp.reshape(matmul(cc2d, hwk), (_HB, _NC, _E)) + hbk[None]
        q_hi = matmul(h_bar + h_rest0, hwq) + hbq

        ph = jnp.zeros((_HB, _NC), f32)
        init_h_hi = None
        fx7 = fy7 = None
        for i in range(_HIGH_STEPS):
            prob = probs_from_q(q_hi, k_high, hv, hmask)
            logits = jnp.log(prob + 1e-10)
            oh = argmax_onehot(logits + gh_full[i, S])
            ph = ph + oh * logits
            hmask = hmask * (1.0 - oh) + oh
            h_hi = row_gather(oh, cc)
            if i == 0:
                init_h_hi = h_hi
            h_rest = matmul(jnp.concatenate([init_h_hi, h_hi], axis=1),
                            wvf) + bvw
            q_hi = matmul(h_bar + h_rest, hwq) + hbq

            cur_cell = row_gather(oh, nc)
            lmask = row_gather(oh, lm)

            h_mean_l = jnp.mean(cur_cell, axis=1)
            h_bar_l = matmul(h_mean_l, wlh) + blh
            cur2d = jnp.reshape(cur_cell, (_HB * _L, _E))
            k_low = jnp.reshape(matmul(cur2d, lwk), (_HB, _L, _E)) + lbk[None]
            q_lo = matmul(h_bar_l + lh_rest0, lwq) + lbq

            need_coords = i >= _HIGH_STEPS - 2
            if need_coords:
                ocx = row_gather(oh, ox)
                ocy = row_gather(oh, oy)

            pl_acc = jnp.zeros((_HB, _NC), f32)
            init_h_l = None
            ohs_list = []
            for s in range(_L):
                probl = probs_from_q(q_lo, k_low, lv, lmask)
                logitsl = jnp.log(probl + 1e-10)
                ohs = argmax_onehot(logitsl + gl_full[i][S, s])
                if i == _HIGH_STEPS - 1:
                    pl_acc = pl_acc + ohs * logitsl
                lmask = lmask * (1.0 - ohs) + ohs
                h = row_gather(ohs, cur_cell)
                if s == 0:
                    init_h_l = h
                h_rest = matmul(jnp.concatenate([init_h_l, h], axis=1),
                                wlvf) + blv
                q_lo = matmul(h_bar_l + h_rest, lwq) + lbq
                if need_coords:
                    ohs_list.append(ohs)

            if i == _HIGH_STEPS - 2:
                fx7 = jnp.sum(ohs_list[-1] * ocx, axis=-1, keepdims=True)
                fy7 = jnp.sum(ohs_list[-1] * ocy, axis=-1, keepdims=True)
            if i == _HIGH_STEPS - 1:
                ohs_stack = jnp.stack(ohs_list, axis=1)
                x = jnp.sum(ohs_stack * ocx[:, None, :], axis=-1)
                y = jnp.sum(ohs_stack * ocy[:, None, :], axis=-1)
                xp = jnp.concatenate([ocx[:, 0:1], x[:, :-1]], axis=1)
                yp = jnp.concatenate([ocy[:, 0:1], y[:, :-1]], axis=1)
                local_sum = jnp.sum(jnp.sqrt(
                    (x - xp) ** 2 + (y - yp) ** 2 + 1e-12))
                ix, iy = ocx[:, 0:1], ocy[:, 0:1]
                cell_rw = jnp.sum(jnp.sqrt(
                    (fx7 - ix) ** 2 + (fy7 - iy) ** 2 + 1e-12))
                total_reward = total_reward + cell_rw + local_sum
                total_lp = total_lp + jnp.sum(pl_acc)
        total_lp = total_lp + jnp.sum(ph)

    out_iota = lax.broadcasted_iota(jnp.int32, (1, _E), 1)
    out_ref[...] = jnp.where(
        out_iota == 0, total_lp,
        jnp.where(out_iota == 1, total_reward, 0.0))


def kernel(node_context, original_data, cell_context, high_mask, low_mask, params):
    gh_np, gl_np = _gumbel_tables()
    f32 = jnp.float32

    def r2(v):
        return jnp.reshape(v, (1, -1)).astype(f32)

    hp, lp = params['high_ptr'], params['low_ptr']
    w_vw, b_vw = params['v_w']
    w_lvw, b_lvw = params['low_v_w']

    args = (
        node_context.astype(f32),
        original_data[..., 0].astype(f32),
        original_data[..., 1].astype(f32),
        cell_context.astype(f32),
        high_mask.astype(f32),
        low_mask.astype(f32),
        params['h_ctx'][0].astype(f32), r2(params['h_ctx'][1]),
        w_vw[:_E].astype(f32), w_vw[_E:].astype(f32), r2(b_vw),
        jnp.reshape(params['init_w'], (2, _E)).astype(f32),
        hp['Wq'].astype(f32), r2(hp['bq']), hp['Wk'].astype(f32),
        r2(hp['bk']), jnp.reshape(hp['V'], (_E, 1)).astype(f32),
        params['low_h_ctx'][0].astype(f32), r2(params['low_h_ctx'][1]),
        w_lvw[:_E].astype(f32), w_lvw[_E:].astype(f32), r2(b_lvw),
        jnp.reshape(params['low_init_w'], (2, _E)).astype(f32),
        lp['Wq'].astype(f32), r2(lp['bq']), lp['Wk'].astype(f32),
        r2(lp['bk']), jnp.reshape(lp['V'], (_E, 1)).astype(f32),
        jnp.asarray(gh_np), jnp.asarray(gl_np),
    )

    out = pl.pallas_call(
        _decoder_kernel,
        out_shape=jax.ShapeDtypeStruct((1, _E), f32),
    )(*args)

    total_log_prob = out[0, 0:1]
    total_reward = out[0, 1:2]
    return total_log_prob, total_reward
